# Initial kernel scaffold; baseline (speedup 1.0000x reference)
import jax
import jax.numpy as jnp
from jax.experimental import pallas as pl


def kernel(tokens, wte_weight, soft_prompt_embeds, W_mean, W_logv, W_l2h):
    raise NotImplementedError("write your pallas kernel here")


# speedup vs baseline: 2.2778x; 2.2778x over previous
import functools

import jax
import jax.numpy as jnp
from jax import lax
from jax.experimental import pallas as pl
from jax.experimental.pallas import tpu as pltpu
from jax.experimental.pallas import tpu_sc as plsc

_HIDDEN = 768
_NP = 8
_B = 4
_S = 2048
_NS = 3
_NW = 32
_TPW = (_B * _S) // _NW
_C = 64
_NCH = _TPW // _C
_ROWS = _NP + _S
_OUT_ROWS = _B * _NS * _ROWS


def _prompt_tc(sp_ref, wm_ref, wv_ref, wl_ref, out_ref):
    dn = (((1,), (1,)), ((), ()))
    sp = sp_ref[...]
    mean = lax.dot_general(sp, wm_ref[...], dn,
                           preferred_element_type=jnp.float32,
                           precision=lax.Precision.HIGHEST)
    logv = lax.dot_general(sp, wv_ref[...], dn,
                           preferred_element_type=jnp.float32,
                           precision=lax.Precision.HIGHEST)
    std = jnp.exp(0.5 * logv)
    for i, sgn in enumerate((-1.0, 0.0, 1.0)):
        z = mean + sgn * std
        out_ref[i * _NP:(i + 1) * _NP, :] = lax.dot_general(
            z, wl_ref[...], dn,
            preferred_element_type=jnp.float32,
            precision=lax.Precision.HIGHEST)


def _sc_body(tok_hbm, wte_hbm, epp_hbm, out_hbm, idx_v, rows_v, epp_v, sem):
    c = lax.axis_index("c")
    s = lax.axis_index("s")
    wid = s * 2 + c
    b = wid // 8
    p = lax.rem(wid, 8)
    pltpu.sync_copy(tok_hbm.at[pl.ds(wid * _NCH, _NCH)], idx_v)
    for j in range(_NCH):
        pltpu.async_copy(wte_hbm.at[idx_v.at[j]], rows_v, sem).wait()
        src_base = p * _TPW + j * _C
        for s_i in range(_NS):
            out_base = (b * _NS + s_i) * _ROWS + _NP + src_base
            pltpu.sync_copy(rows_v, out_hbm.at[pl.ds(out_base, _C)])

    @pl.when(wid < _B * _NS)
    def _():
        r = lax.rem(wid, _NS)
        pltpu.sync_copy(epp_hbm.at[pl.ds(r * _NP, _NP)], epp_v)
        pltpu.sync_copy(epp_v, out_hbm.at[pl.ds(wid * _ROWS, _NP)])


_sc_gather = functools.partial(
    pl.kernel,
    mesh=plsc.VectorSubcoreMesh(core_axis_name="c", subcore_axis_name="s"),
    out_type=jax.ShapeDtypeStruct((_OUT_ROWS, _HIDDEN), jnp.float32),
    scratch_types=[
        pltpu.VMEM((_NCH, _C), jnp.int32),
        pltpu.VMEM((_C, _HIDDEN), jnp.float32),
        pltpu.VMEM((_NP, _HIDDEN), jnp.float32),
        pltpu.SemaphoreType.DMA,
    ],
)(_sc_body)


def kernel(tokens, wte_weight, soft_prompt_embeds, W_mean, W_logv, W_l2h):
    epp = pl.pallas_call(
        _prompt_tc,
        out_shape=jax.ShapeDtypeStruct((_NS * _NP, _HIDDEN), jnp.float32),
    )(soft_prompt_embeds, W_mean, W_logv, W_l2h)
    tok = tokens.astype(jnp.int32).reshape(_NW * _NCH, _C)
    out = _sc_gather(tok, wte_weight, epp)
    return out.reshape(_B * _NS, _ROWS, _HIDDEN)


# speedup vs baseline: 2.4005x; 1.0539x over previous
import functools

import jax
import jax.numpy as jnp
from jax import lax
from jax.experimental import pallas as pl
from jax.experimental.pallas import tpu as pltpu
from jax.experimental.pallas import tpu_sc as plsc

_HIDDEN = 768
_NP = 8
_B = 4
_S = 2048
_NS = 3
_NW = 32
_TPW = (_B * _S) // _NW
_C = 64
_NCH = _TPW // _C
_ROWS = _NP + _S
_OUT_ROWS = _B * _NS * _ROWS


def _prompt_tc(sp_ref, wm_ref, wv_ref, wl_ref, out_ref):
    dn = (((1,), (1,)), ((), ()))
    sp = sp_ref[...]
    mean = lax.dot_general(sp, wm_ref[...], dn,
                           preferred_element_type=jnp.float32,
                           precision=lax.Precision.HIGHEST)
    logv = lax.dot_general(sp, wv_ref[...], dn,
                           preferred_element_type=jnp.float32,
                           precision=lax.Precision.HIGHEST)
    std = jnp.exp(0.5 * logv)
    for i, sgn in enumerate((-1.0, 0.0, 1.0)):
        z = mean + sgn * std
        out_ref[i * _NP:(i + 1) * _NP, :] = lax.dot_general(
            z, wl_ref[...], dn,
            preferred_element_type=jnp.float32,
            precision=lax.Precision.HIGHEST)


def _sc_body(tok_hbm, wte_hbm, epp_hbm, out_hbm,
             idx_v, rows0, rows1, epp_v, gsem0, gsem1, wsem0, wsem1):
    c = lax.axis_index("c")
    s = lax.axis_index("s")
    wid = s * 2 + c
    b = wid // 8
    p = lax.rem(wid, 8)
    bufs = (rows0, rows1)
    gsems = (gsem0, gsem1)
    wsems = (wsem0, wsem1)
    pltpu.sync_copy(tok_hbm.at[pl.ds(wid * _NCH, _NCH)], idx_v)
    g = [None] * _NCH
    w = [[] for _ in range(_NCH)]
    g[0] = pltpu.async_copy(wte_hbm.at[idx_v.at[0]], bufs[0], gsems[0])

    @pl.when(wid < _B * _NS)
    def _():
        r = lax.rem(wid, _NS)
        pltpu.sync_copy(epp_hbm.at[pl.ds(r * _NP, _NP)], epp_v)
        pltpu.sync_copy(epp_v, out_hbm.at[pl.ds(wid * _ROWS, _NP)])

    for j in range(_NCH):
        bj = j % 2
        if j + 1 < _NCH:
            if j >= 1:
                for d in w[j - 1]:
                    d.wait()
            g[j + 1] = pltpu.async_copy(
                wte_hbm.at[idx_v.at[j + 1]], bufs[1 - bj], gsems[1 - bj])
        g[j].wait()
        src_base = p * _TPW + j * _C
        for s_i in range(_NS):
            out_base = (b * _NS + s_i) * _ROWS + _NP + src_base
            w[j].append(pltpu.async_copy(
                bufs[bj], out_hbm.at[pl.ds(out_base, _C)], wsems[bj]))
    for d in w[_NCH - 2]:
        d.wait()
    for d in w[_NCH - 1]:
        d.wait()


_sc_gather = functools.partial(
    pl.kernel,
    mesh=plsc.VectorSubcoreMesh(core_axis_name="c", subcore_axis_name="s"),
    out_type=jax.ShapeDtypeStruct((_OUT_ROWS, _HIDDEN), jnp.float32),
    scratch_types=[
        pltpu.VMEM((_NCH, _C), jnp.int32),
        pltpu.VMEM((_C, _HIDDEN), jnp.float32),
        pltpu.VMEM((_C, _HIDDEN), jnp.float32),
        pltpu.VMEM((_NP, _HIDDEN), jnp.float32),
        pltpu.SemaphoreType.DMA,
        pltpu.SemaphoreType.DMA,
        pltpu.SemaphoreType.DMA,
        pltpu.SemaphoreType.DMA,
    ],
)(_sc_body)


def kernel(tokens, wte_weight, soft_prompt_embeds, W_mean, W_logv, W_l2h):
    epp = pl.pallas_call(
        _prompt_tc,
        out_shape=jax.ShapeDtypeStruct((_NS * _NP, _HIDDEN), jnp.float32),
    )(soft_prompt_embeds, W_mean, W_logv, W_l2h)
    tok = tokens.astype(jnp.int32).reshape(_NW * _NCH, _C)
    out = _sc_gather(tok, wte_weight, epp)
    return out.reshape(_B * _NS, _ROWS, _HIDDEN)
